# Initial kernel scaffold; baseline (speedup 1.0000x reference)
import jax
import jax.numpy as jnp
from jax.experimental import pallas as pl


def kernel(y, e, f):
    raise NotImplementedError("write your pallas kernel here")


# speedup vs baseline: 95.1017x; 95.1017x over previous
import functools

import jax
import jax.numpy as jnp
import numpy as np
from jax import lax
from jax.experimental import pallas as pl
from jax.experimental.pallas import tpu as pltpu
from jax.experimental.pallas import tpu_sc as plsc

_NB = 4
_NV = 10000
_DY = 128
_DE = 64
_NW = 32
_ROWS = _NB * _NV
_RPW = _ROWS // _NW
_NCHUNK = 10
_C = _RPW // _NCHUNK

def _perm_and_idx():
    s = jax.random.permutation(jax.random.key(42), jnp.arange(_NV, dtype=jnp.int32))
    idx = (jnp.arange(_NB, dtype=jnp.int32)[:, None] * _NV + s[None, :]).reshape(
        _NW, _NCHUNK, _C
    )
    return s, idx


@functools.lru_cache(maxsize=1)
def _build():
    mesh = plsc.VectorSubcoreMesh(core_axis_name="c", subcore_axis_name="s")

    @functools.partial(
        pl.kernel,
        out_type=(
            jax.ShapeDtypeStruct((_ROWS, _DY), jnp.float32),
            jax.ShapeDtypeStruct((_ROWS, _DE), jnp.int32),
            jax.ShapeDtypeStruct((_ROWS, _DE), jnp.float32),
        ),
        mesh=mesh,
        compiler_params=pltpu.CompilerParams(
            use_tc_tiling_on_sc=False, needs_layout_passes=False
        ),
        scratch_types=[
            pltpu.VMEM((_NCHUNK, _C), jnp.int32),
            pltpu.VMEM((_NV,), jnp.int32),
            pltpu.VMEM((_C, _DY), jnp.float32),
            pltpu.VMEM((_C, _DE), jnp.int32),
            pltpu.VMEM((_C, _DE), jnp.float32),
            pltpu.SemaphoreType.DMA,
            pltpu.SemaphoreType.DMA,
        ],
    )
    def _shuffle(y_hbm, e_hbm, f_hbm, idx_hbm, s_hbm,
                 y_out, e_out, f_out,
                 idx_v, s_v, ybuf, ebuf, fbuf, gsem, ssem):
        wid = lax.axis_index("s") * 2 + lax.axis_index("c")
        pltpu.sync_copy(idx_hbm.at[wid], idx_v)
        pltpu.sync_copy(s_hbm, s_v)
        row0 = wid * _RPW

        for c in range(_NCHUNK):
            gy = pltpu.async_copy(y_hbm.at[idx_v.at[c]], ybuf, gsem)
            ge = pltpu.async_copy(e_hbm.at[idx_v.at[c]], ebuf, gsem)
            gf = pltpu.async_copy(f_hbm.at[idx_v.at[c]], fbuf, gsem)
            gy.wait()
            ge.wait()
            gf.wait()

            def _remap(i, _):
                for k in range(_DE // 16):
                    vals = ebuf[i, pl.ds(k * 16, 16)]
                    ebuf[i, pl.ds(k * 16, 16)] = plsc.load_gather(s_v, [vals])
                return 0

            lax.fori_loop(0, _C, _remap, 0)

            dst = row0 + c * _C
            sy = pltpu.async_copy(ybuf, y_out.at[pl.ds(dst, _C)], ssem)
            se = pltpu.async_copy(ebuf, e_out.at[pl.ds(dst, _C)], ssem)
            sf = pltpu.async_copy(fbuf, f_out.at[pl.ds(dst, _C)], ssem)
            sy.wait()
            se.wait()
            sf.wait()

    return _shuffle


def kernel(y, e, f):
    y2d = y.reshape(_ROWS, _DY)
    e2d = e.reshape(_ROWS, _DE)
    f2d = f.reshape(_ROWS, _DE)
    s, idx = _perm_and_idx()
    y_o, e_o, f_o = _build()(y2d, e2d, f2d, idx, s)
    return (
        y_o.reshape(_NB, _NV, _DY),
        e_o.reshape(_NB, _NV, 4, 16),
        f_o.reshape(_NB, _NV, 4, 16),
        s,
        s,
    )
